# Initial kernel scaffold; baseline (speedup 1.0000x reference)
import jax
import jax.numpy as jnp
from jax.experimental import pallas as pl


def kernel(x, edge_index, edge_weight, W_l, b_l, W_r, W_out, b_out):
    raise NotImplementedError("write your pallas kernel here")


# speedup vs baseline: 52.1998x; 52.1998x over previous
import jax
import jax.numpy as jnp
from jax import lax
from jax.experimental import pallas as pl
from jax.experimental.pallas import tpu as pltpu
from jax.experimental.pallas import tpu_sc as plsc

N_NODES = 100000
N_EDGES = 3200000
D_IN = 17

BLK = 2048
GRID = 49
N2 = BLK * GRID
NC, NS = 2, 16
NW = NC * NS
EP_TILE = N_EDGES // NW
K = 10000
NCHUNK = EP_TILE // K
NROW_T = N2 // NS


def _proj_body(x_ref, wl_ref, wr_ref, p_ref, r_ref):
    xb = x_ref[...]
    p_ref[...] = jnp.sum(xb * wl_ref[...], axis=1, keepdims=True)
    r_ref[...] = jnp.sum(xb * wr_ref[...], axis=1, keepdims=True)


def _proj(x, W_l, W_r):
    return pl.pallas_call(
        _proj_body,
        grid=(GRID,),
        in_specs=[
            pl.BlockSpec((BLK, D_IN), lambda i: (i, 0)),
            pl.BlockSpec((1, D_IN), lambda i: (0, 0)),
            pl.BlockSpec((1, D_IN), lambda i: (0, 0)),
        ],
        out_specs=[
            pl.BlockSpec((BLK, 1), lambda i: (i, 0)),
            pl.BlockSpec((BLK, 1), lambda i: (i, 0)),
        ],
        out_shape=[
            jax.ShapeDtypeStruct((N2, 1), jnp.float32),
            jax.ShapeDtypeStruct((N2, 1), jnp.float32),
        ],
    )(x, W_l, W_r)


def _sc_body(src_hbm, dst_hbm, p_hbm, z_hbm, ones_hbm, s_out, c_out,
             src_v, dst_v, vals_v, ones_v, stage_v, p_s, s_s, c_s, sem):
    c = lax.axis_index("c")
    s = lax.axis_index("s")
    wid = c * NS + s
    row = s * NROW_T

    pltpu.sync_copy(z_hbm, stage_v)
    pltpu.sync_copy(stage_v, s_s.at[pl.ds(row, NROW_T)])
    pltpu.sync_copy(stage_v, c_s.at[pl.ds(row, NROW_T)])
    pltpu.sync_copy(p_hbm.at[pl.ds(row, NROW_T)], stage_v)
    pltpu.sync_copy(stage_v, p_s.at[pl.ds(row, NROW_T)])
    pltpu.sync_copy(ones_hbm, ones_v)
    plsc.subcore_barrier()

    for j in range(NCHUNK):
        base = wid * EP_TILE + j * K
        pltpu.sync_copy(src_hbm.at[pl.ds(base, K)], src_v)
        pltpu.sync_copy(dst_hbm.at[pl.ds(base, K)], dst_v)
        pltpu.async_copy(p_s.at[src_v], vals_v, sem).wait()
        pltpu.sync_copy(vals_v, s_s.at[dst_v], add=True)
        pltpu.sync_copy(ones_v, c_s.at[dst_v], add=True)

    plsc.subcore_barrier()
    off = c * N2 + row
    pltpu.sync_copy(s_s.at[pl.ds(row, NROW_T)], stage_v)
    pltpu.sync_copy(stage_v, s_out.at[pl.ds(off, NROW_T)])
    pltpu.sync_copy(c_s.at[pl.ds(row, NROW_T)], stage_v)
    pltpu.sync_copy(stage_v, c_out.at[pl.ds(off, NROW_T)])


def _segment_acc(src, dst, p, z, ones):
    mesh = plsc.VectorSubcoreMesh(core_axis_name="c", subcore_axis_name="s",
                                  num_cores=NC, num_subcores=NS)
    f = pl.kernel(
        _sc_body,
        out_type=[
            jax.ShapeDtypeStruct((NC * N2,), jnp.float32),
            jax.ShapeDtypeStruct((NC * N2,), jnp.float32),
        ],
        mesh=mesh,
        scratch_types=[
            pltpu.VMEM((K,), jnp.int32),
            pltpu.VMEM((K,), jnp.int32),
            pltpu.VMEM((K,), jnp.float32),
            pltpu.VMEM((K,), jnp.float32),
            pltpu.VMEM((NROW_T,), jnp.float32),
            pltpu.VMEM_SHARED((N2,), jnp.float32),
            pltpu.VMEM_SHARED((N2,), jnp.float32),
            pltpu.VMEM_SHARED((N2,), jnp.float32),
            pltpu.SemaphoreType.DMA,
        ],
    )
    return f(src, dst, p, z, ones)


def _combine_body(s0_ref, s1_ref, c0_ref, c1_ref, r_ref, scal_ref, out_ref):
    ssum = s0_ref[...] + s1_ref[...]
    cnt = c0_ref[...] + c1_ref[...]
    mean = ssum / jnp.maximum(cnt, 1.0)
    h = mean + scal_ref[0, 0] + r_ref[...]
    h = jnp.where(h > 0, h, jnp.exp(h) - 1.0)
    out_ref[...] = h * scal_ref[0, 1] + scal_ref[0, 2]


def _combine(s2, c2, r, scal):
    spec0 = pl.BlockSpec((BLK, 1), lambda i: (i, 0))
    spec1 = pl.BlockSpec((BLK, 1), lambda i: (i + GRID, 0))
    return pl.pallas_call(
        _combine_body,
        grid=(GRID,),
        in_specs=[
            spec0, spec1, spec0, spec1, spec0,
            pl.BlockSpec(memory_space=pltpu.SMEM),
        ],
        out_specs=pl.BlockSpec((BLK, 1), lambda i: (i, 0)),
        out_shape=jax.ShapeDtypeStruct((N2, 1), jnp.float32),
    )(s2, s2, c2, c2, r, scal)


def kernel(x, edge_index, edge_weight, W_l, b_l, W_r, W_out, b_out):
    ei = edge_index.astype(jnp.int32)
    p, r = _proj(x, W_l, W_r)
    z = jnp.zeros((NROW_T,), jnp.float32)
    ones = jnp.ones((K,), jnp.float32)
    s2, c2 = _segment_acc(ei[0], ei[1], p.reshape(N2), z, ones)
    scal = jnp.concatenate([b_l, W_out[0], b_out]).reshape(1, 3)
    out = _combine(s2.reshape(NC * N2, 1), c2.reshape(NC * N2, 1), r, scal)
    return out[:N_NODES]


# speedup vs baseline: 115.1811x; 2.2065x over previous
import jax
import jax.numpy as jnp
from jax import lax
from jax.experimental import pallas as pl
from jax.experimental.pallas import tpu as pltpu
from jax.experimental.pallas import tpu_sc as plsc

N_NODES = 100000
N_EDGES = 3200000
D_IN = 17

BLK = 2048
GRID = 49
N2 = BLK * GRID
R = N2 // 128
RB = 112
NC, NS = 2, 16
NW = NC * NS
EP_TILE = N_EDGES // NW
K = 10000
NCHUNK = EP_TILE // K
NROW_T = N2 // NS


def _proj_body(x_ref, w2_ref, p_ref, r_ref):
    pr = jnp.dot(x_ref[...], w2_ref[...],
                 preferred_element_type=jnp.float32)
    p_ref[...] = pr[:, 0].reshape(BLK // 128, 128)
    r_ref[...] = pr[:, 1].reshape(BLK // 128, 128)


def _proj(x, w2):
    return pl.pallas_call(
        _proj_body,
        grid=(GRID,),
        in_specs=[
            pl.BlockSpec((BLK, D_IN), lambda i: (i, 0)),
            pl.BlockSpec((D_IN, 2), lambda i: (0, 0)),
        ],
        out_specs=[
            pl.BlockSpec((BLK // 128, 128), lambda i: (i, 0)),
            pl.BlockSpec((BLK // 128, 128), lambda i: (i, 0)),
        ],
        out_shape=[
            jax.ShapeDtypeStruct((R, 128), jnp.float32),
            jax.ShapeDtypeStruct((R, 128), jnp.float32),
        ],
    )(x, w2)


def _sc_body(ei_hbm, p_hbm, z_hbm, ones_hbm, s_out, c_out,
             src_v, dst_v, vals_v, ones_v, stage_v, p_s, s_s, c_s, sem):
    c = lax.axis_index("c")
    s = lax.axis_index("s")
    wid = c * NS + s
    row = s * NROW_T

    pltpu.sync_copy(z_hbm, stage_v)
    pltpu.sync_copy(stage_v, s_s.at[pl.ds(row, NROW_T)])
    pltpu.sync_copy(stage_v, c_s.at[pl.ds(row, NROW_T)])
    pltpu.sync_copy(p_hbm.at[pl.ds(row, NROW_T)], stage_v)
    pltpu.sync_copy(stage_v, p_s.at[pl.ds(row, NROW_T)])
    pltpu.sync_copy(ones_hbm, ones_v)
    plsc.subcore_barrier()

    for j in range(NCHUNK):
        base = wid * EP_TILE + j * K
        pltpu.sync_copy(ei_hbm.at[pl.ds(base, K)], src_v)
        pltpu.sync_copy(ei_hbm.at[pl.ds(N_EDGES + base, K)], dst_v)
        pltpu.async_copy(p_s.at[src_v], vals_v, sem).wait()
        pltpu.sync_copy(vals_v, s_s.at[dst_v], add=True)
        pltpu.sync_copy(ones_v, c_s.at[dst_v], add=True)

    plsc.subcore_barrier()
    off = c * N2 + row
    pltpu.sync_copy(s_s.at[pl.ds(row, NROW_T)], stage_v)
    pltpu.sync_copy(stage_v, s_out.at[pl.ds(off, NROW_T)])
    pltpu.sync_copy(c_s.at[pl.ds(row, NROW_T)], stage_v)
    pltpu.sync_copy(stage_v, c_out.at[pl.ds(off, NROW_T)])


def _segment_acc(ei2, p, z, ones):
    mesh = plsc.VectorSubcoreMesh(core_axis_name="c", subcore_axis_name="s",
                                  num_cores=NC, num_subcores=NS)
    f = pl.kernel(
        _sc_body,
        out_type=[
            jax.ShapeDtypeStruct((NC * N2,), jnp.float32),
            jax.ShapeDtypeStruct((NC * N2,), jnp.float32),
        ],
        mesh=mesh,
        scratch_types=[
            pltpu.VMEM((K,), jnp.int32),
            pltpu.VMEM((K,), jnp.int32),
            pltpu.VMEM((K,), jnp.float32),
            pltpu.VMEM((K,), jnp.float32),
            pltpu.VMEM((NROW_T,), jnp.float32),
            pltpu.VMEM_SHARED((N2,), jnp.float32),
            pltpu.VMEM_SHARED((N2,), jnp.float32),
            pltpu.VMEM_SHARED((N2,), jnp.float32),
            pltpu.SemaphoreType.DMA,
        ],
    )
    return f(ei2, p, z, ones)


def _combine_body(s0_ref, s1_ref, c0_ref, c1_ref, r_ref, scal_ref, out_ref):
    ssum = s0_ref[...] + s1_ref[...]
    cnt = c0_ref[...] + c1_ref[...]
    mean = ssum / jnp.maximum(cnt, 1.0)
    h = mean + scal_ref[0, 0] + r_ref[...]
    h = jnp.where(h > 0, h, jnp.exp(h) - 1.0)
    out_ref[...] = h * scal_ref[0, 1] + scal_ref[0, 2]


def _combine(s2m, c2m, r2d, scal):
    spec0 = pl.BlockSpec((RB, 128), lambda i: (i, 0))
    spec1 = pl.BlockSpec((RB, 128), lambda i: (i + R // RB, 0))
    return pl.pallas_call(
        _combine_body,
        grid=(R // RB,),
        in_specs=[
            spec0, spec1, spec0, spec1, spec0,
            pl.BlockSpec(memory_space=pltpu.SMEM),
        ],
        out_specs=pl.BlockSpec((RB, 128), lambda i: (i, 0)),
        out_shape=jax.ShapeDtypeStruct((R, 128), jnp.float32),
    )(s2m, s2m, c2m, c2m, r2d, scal)


def kernel(x, edge_index, edge_weight, W_l, b_l, W_r, W_out, b_out):
    ei2 = edge_index.astype(jnp.int32).reshape(2 * N_EDGES)
    w2 = jnp.concatenate([W_l, W_r], axis=0).T
    p2d, r2d = _proj(x, w2)
    z = jnp.zeros((NROW_T,), jnp.float32)
    ones = jnp.ones((K,), jnp.float32)
    s2, c2 = _segment_acc(ei2, p2d.reshape(N2), z, ones)
    scal = jnp.concatenate([b_l, W_out[0], b_out]).reshape(1, 3)
    out2d = _combine(s2.reshape(2 * R, 128), c2.reshape(2 * R, 128),
                     r2d, scal)
    return out2d.reshape(N2, 1)[:N_NODES]


# speedup vs baseline: 121.5932x; 1.0557x over previous
import jax
import jax.numpy as jnp
from jax import lax
from jax.experimental import pallas as pl
from jax.experimental.pallas import tpu as pltpu
from jax.experimental.pallas import tpu_sc as plsc

N_NODES = 100000
N_EDGES = 3200000
D_IN = 17

BLK = 2048
GRID = 49
N2 = BLK * GRID
R = N2 // 128
RB = 112
NC, NS = 2, 16
NW = NC * NS
EP_TILE = 99968
K = 9088
NCHUNK = EP_TILE // K
TAIL_BASE = NW * EP_TILE
TAIL_N = N_EDGES - TAIL_BASE
NROW_T = N2 // NS


def _proj_body(x_ref, w2_ref, p_ref, r_ref):
    pr = jnp.dot(x_ref[...], w2_ref[...],
                 preferred_element_type=jnp.float32)
    p_ref[...] = pr[:, 0].reshape(BLK // 128, 128)
    r_ref[...] = pr[:, 1].reshape(BLK // 128, 128)


def _proj(x, w2):
    return pl.pallas_call(
        _proj_body,
        grid=(GRID,),
        in_specs=[
            pl.BlockSpec((BLK, D_IN), lambda i: (i, 0)),
            pl.BlockSpec((D_IN, 2), lambda i: (0, 0)),
        ],
        out_specs=[
            pl.BlockSpec((BLK // 128, 128), lambda i: (i, 0)),
            pl.BlockSpec((BLK // 128, 128), lambda i: (i, 0)),
        ],
        out_shape=[
            jax.ShapeDtypeStruct((R, 128), jnp.float32),
            jax.ShapeDtypeStruct((R, 128), jnp.float32),
        ],
    )(x, w2)


def _deinterleave(eib, src_v, dst_v, n):
    def body(i, _):
        off = pl.ds(pl.multiple_of(i * 16, 16), 16)
        src_v[off] = eib[0, off]
        dst_v[off] = eib[1, off]
        return 0

    lax.fori_loop(0, n // 16, body, 0)


def _sc_body(ei_hbm, p_hbm, z_hbm, ones_hbm, s_out, c_out,
             ei_v0, ei_v1, src_v0, src_v1, dst_v0, dst_v1, vals_v0,
             vals_v1, ones_v, stage_v, src_t, dst_t, vals_t,
             p_s, s_s, c_s, sem_i0, sem_i1, sem_g, sem_s, sem_c):
    c = lax.axis_index("c")
    s = lax.axis_index("s")
    wid = c * NS + s
    row = s * NROW_T
    ei_bufs = (ei_v0, ei_v1)
    src_bufs = (src_v0, src_v1)
    dst_bufs = (dst_v0, dst_v1)
    vals_bufs = (vals_v0, vals_v1)
    sems = (sem_i0, sem_i1)

    def idx_start(j):
        base = wid * EP_TILE + j * K
        return pltpu.async_copy(ei_hbm.at[:, pl.ds(base, K)],
                                ei_bufs[j % 2], sems[j % 2])

    d_idx = idx_start(0)

    pltpu.sync_copy(z_hbm, stage_v)
    pltpu.sync_copy(stage_v, s_s.at[pl.ds(row, NROW_T)])
    pltpu.sync_copy(stage_v, c_s.at[pl.ds(row, NROW_T)])
    pltpu.sync_copy(p_hbm.at[pl.ds(row, NROW_T)], stage_v)
    pltpu.sync_copy(stage_v, p_s.at[pl.ds(row, NROW_T)])
    pltpu.sync_copy(ones_hbm, ones_v)
    plsc.subcore_barrier()

    d_ss = d_sc = None
    for j in range(NCHUNK):
        b = j % 2
        d_idx.wait()
        if j + 1 < NCHUNK:
            d_idx = idx_start(j + 1)
        if j >= 2:
            d_ss2.wait()
            d_sc2.wait()
        _deinterleave(ei_bufs[b], src_bufs[b], dst_bufs[b], K)
        pltpu.async_copy(p_s.at[src_bufs[b]], vals_bufs[b], sem_g).wait()
        d_ss2, d_sc2 = d_ss, d_sc
        d_ss = pltpu.async_copy(vals_bufs[b], s_s.at[dst_bufs[b]],
                                sem_s, add=True)
        d_sc = pltpu.async_copy(ones_v, c_s.at[dst_bufs[b]],
                                sem_c, add=True)
    d_ss2.wait()
    d_sc2.wait()
    d_ss.wait()
    d_sc.wait()

    @pl.when(wid < TAIL_N // 128)
    def _tail():
        pltpu.sync_copy(ei_hbm.at[:, pl.ds(TAIL_BASE + wid * 128, 128)],
                        ei_v0.at[:, pl.ds(0, 128)])
        _deinterleave(ei_v0, src_t, dst_t, 128)
        pltpu.async_copy(p_s.at[src_t], vals_t, sem_g).wait()
        pltpu.sync_copy(vals_t, s_s.at[dst_t], add=True)
        pltpu.sync_copy(ones_v.at[pl.ds(0, 128)], c_s.at[dst_t], add=True)

    plsc.subcore_barrier()
    off = c * N2 + row
    pltpu.sync_copy(s_s.at[pl.ds(row, NROW_T)], stage_v)
    pltpu.sync_copy(stage_v, s_out.at[pl.ds(off, NROW_T)])
    pltpu.sync_copy(c_s.at[pl.ds(row, NROW_T)], stage_v)
    pltpu.sync_copy(stage_v, c_out.at[pl.ds(off, NROW_T)])


def _segment_acc(ei, p, z, ones):
    mesh = plsc.VectorSubcoreMesh(core_axis_name="c", subcore_axis_name="s",
                                  num_cores=NC, num_subcores=NS)
    f = pl.kernel(
        _sc_body,
        out_type=[
            jax.ShapeDtypeStruct((NC * N2,), jnp.float32),
            jax.ShapeDtypeStruct((NC * N2,), jnp.float32),
        ],
        mesh=mesh,
        scratch_types=[
            pltpu.VMEM((2, K), jnp.int32),
            pltpu.VMEM((2, K), jnp.int32),
            pltpu.VMEM((K,), jnp.int32),
            pltpu.VMEM((K,), jnp.int32),
            pltpu.VMEM((K,), jnp.int32),
            pltpu.VMEM((K,), jnp.int32),
            pltpu.VMEM((K,), jnp.float32),
            pltpu.VMEM((K,), jnp.float32),
            pltpu.VMEM((K,), jnp.float32),
            pltpu.VMEM((NROW_T,), jnp.float32),
            pltpu.VMEM((128,), jnp.int32),
            pltpu.VMEM((128,), jnp.int32),
            pltpu.VMEM((128,), jnp.float32),
            pltpu.VMEM_SHARED((N2,), jnp.float32),
            pltpu.VMEM_SHARED((N2,), jnp.float32),
            pltpu.VMEM_SHARED((N2,), jnp.float32),
            pltpu.SemaphoreType.DMA,
            pltpu.SemaphoreType.DMA,
            pltpu.SemaphoreType.DMA,
            pltpu.SemaphoreType.DMA,
            pltpu.SemaphoreType.DMA,
        ],
    )
    return f(ei, p, z, ones)


def _combine_body(s0_ref, s1_ref, c0_ref, c1_ref, r_ref, scal_ref, out_ref):
    ssum = s0_ref[...] + s1_ref[...]
    cnt = c0_ref[...] + c1_ref[...]
    mean = ssum / jnp.maximum(cnt, 1.0)
    h = mean + scal_ref[0, 0] + r_ref[...]
    h = jnp.where(h > 0, h, jnp.exp(h) - 1.0)
    out_ref[...] = h * scal_ref[0, 1] + scal_ref[0, 2]


def _combine(s2m, c2m, r2d, scal):
    spec0 = pl.BlockSpec((RB, 128), lambda i: (i, 0))
    spec1 = pl.BlockSpec((RB, 128), lambda i: (i + R // RB, 0))
    return pl.pallas_call(
        _combine_body,
        grid=(R // RB,),
        in_specs=[
            spec0, spec1, spec0, spec1, spec0,
            pl.BlockSpec(memory_space=pltpu.SMEM),
        ],
        out_specs=pl.BlockSpec((RB, 128), lambda i: (i, 0)),
        out_shape=jax.ShapeDtypeStruct((R, 128), jnp.float32),
    )(s2m, s2m, c2m, c2m, r2d, scal)


def kernel(x, edge_index, edge_weight, W_l, b_l, W_r, W_out, b_out):
    ei = edge_index.astype(jnp.int32)
    w2 = jnp.concatenate([W_l, W_r], axis=0).T
    p2d, r2d = _proj(x, w2)
    z = jnp.zeros((NROW_T,), jnp.float32)
    ones = jnp.ones((K,), jnp.float32)
    s2, c2 = _segment_acc(ei, p2d.reshape(N2), z, ones)
    scal = jnp.concatenate([b_l, W_out[0], b_out]).reshape(1, 3)
    out2d = _combine(s2.reshape(2 * R, 128), c2.reshape(2 * R, 128),
                     r2d, scal)
    return out2d.reshape(N2, 1)[:N_NODES]


# speedup vs baseline: 151.6022x; 1.2468x over previous
import jax
import jax.numpy as jnp
from jax import lax
from jax.experimental import pallas as pl
from jax.experimental.pallas import tpu as pltpu
from jax.experimental.pallas import tpu_sc as plsc

N_NODES = 100000
N_EDGES = 3200000
D_IN = 17

BLK = 2048
GRID = 49
N2 = BLK * GRID
R = N2 // 128
RB = 112
NC, NS = 2, 16
NW = NC * NS
EP_TILE = 99968
K = 9088
NCHUNK = EP_TILE // K
TAIL_BASE = NW * EP_TILE
TAIL_N = N_EDGES - TAIL_BASE
NROW_T = N2 // NS


def _proj_body(xt_ref, w2_ref, p_ref, r_ref):
    pr = lax.dot_general(w2_ref[...], xt_ref[...],
                         dimension_numbers=(((0,), (0,)), ((), ())),
                         preferred_element_type=jnp.float32)
    p_ref[...] = pr[0].reshape(BLK // 128, 128)
    r_ref[...] = pr[1].reshape(BLK // 128, 128)


def _proj(xt, w2):
    return pl.pallas_call(
        _proj_body,
        grid=(GRID,),
        in_specs=[
            pl.BlockSpec((D_IN, BLK), lambda i: (0, i)),
            pl.BlockSpec((D_IN, 2), lambda i: (0, 0)),
        ],
        out_specs=[
            pl.BlockSpec((BLK // 128, 128), lambda i: (i, 0)),
            pl.BlockSpec((BLK // 128, 128), lambda i: (i, 0)),
        ],
        out_shape=[
            jax.ShapeDtypeStruct((R, 128), jnp.float32),
            jax.ShapeDtypeStruct((R, 128), jnp.float32),
        ],
    )(xt, w2)


def _deinterleave(eib, src_v, dst_v, n):
    def body(i, _):
        off = pl.ds(pl.multiple_of(i * 16, 16), 16)
        src_v[off] = eib[0, off]
        dst_v[off] = eib[1, off]
        return 0

    lax.fori_loop(0, n // 16, body, 0)


def _sc_body(ei_hbm, p_hbm, z_hbm, ones_hbm, s_out, c_out,
             ei_v0, ei_v1, src_v0, src_v1, dst_v0, dst_v1, vals_v0,
             vals_v1, ones_v, stage_v, src_t, dst_t, vals_t,
             p_s, s_s, c_s, sem_i0, sem_i1, sem_g, sem_s, sem_c):
    c = lax.axis_index("c")
    s = lax.axis_index("s")
    wid = c * NS + s
    row = s * NROW_T
    ei_bufs = (ei_v0, ei_v1)
    src_bufs = (src_v0, src_v1)
    dst_bufs = (dst_v0, dst_v1)
    vals_bufs = (vals_v0, vals_v1)
    sems = (sem_i0, sem_i1)

    def idx_start(j):
        base = wid * EP_TILE + j * K
        return pltpu.async_copy(ei_hbm.at[:, pl.ds(base, K)],
                                ei_bufs[j % 2], sems[j % 2])

    d_idx = idx_start(0)

    pltpu.sync_copy(z_hbm, stage_v)
    pltpu.sync_copy(stage_v, s_s.at[pl.ds(row, NROW_T)])
    pltpu.sync_copy(stage_v, c_s.at[pl.ds(row, NROW_T)])
    pltpu.sync_copy(p_hbm.at[pl.ds(row, NROW_T)], stage_v)
    pltpu.sync_copy(stage_v, p_s.at[pl.ds(row, NROW_T)])
    pltpu.sync_copy(ones_hbm, ones_v)
    plsc.subcore_barrier()

    d_ss = d_sc = None
    for j in range(NCHUNK):
        b = j % 2
        d_idx.wait()
        if j + 1 < NCHUNK:
            d_idx = idx_start(j + 1)
        if j >= 2:
            d_ss2.wait()
            d_sc2.wait()
        _deinterleave(ei_bufs[b], src_bufs[b], dst_bufs[b], K)
        pltpu.async_copy(p_s.at[src_bufs[b]], vals_bufs[b], sem_g).wait()
        d_ss2, d_sc2 = d_ss, d_sc
        d_ss = pltpu.async_copy(vals_bufs[b], s_s.at[dst_bufs[b]],
                                sem_s, add=True)
        d_sc = pltpu.async_copy(ones_v, c_s.at[dst_bufs[b]],
                                sem_c, add=True)
    d_ss2.wait()
    d_sc2.wait()
    d_ss.wait()
    d_sc.wait()

    @pl.when(wid < TAIL_N // 128)
    def _tail():
        pltpu.sync_copy(ei_hbm.at[:, pl.ds(TAIL_BASE + wid * 128, 128)],
                        ei_v0.at[:, pl.ds(0, 128)])
        _deinterleave(ei_v0, src_t, dst_t, 128)
        pltpu.async_copy(p_s.at[src_t], vals_t, sem_g).wait()
        pltpu.sync_copy(vals_t, s_s.at[dst_t], add=True)
        pltpu.sync_copy(ones_v.at[pl.ds(0, 128)], c_s.at[dst_t], add=True)

    plsc.subcore_barrier()
    off = c * N2 + row
    pltpu.sync_copy(s_s.at[pl.ds(row, NROW_T)], stage_v)
    pltpu.sync_copy(stage_v, s_out.at[pl.ds(off, NROW_T)])
    pltpu.sync_copy(c_s.at[pl.ds(row, NROW_T)], stage_v)
    pltpu.sync_copy(stage_v, c_out.at[pl.ds(off, NROW_T)])


def _segment_acc(ei, p, z, ones):
    mesh = plsc.VectorSubcoreMesh(core_axis_name="c", subcore_axis_name="s",
                                  num_cores=NC, num_subcores=NS)
    f = pl.kernel(
        _sc_body,
        out_type=[
            jax.ShapeDtypeStruct((NC * N2,), jnp.float32),
            jax.ShapeDtypeStruct((NC * N2,), jnp.float32),
        ],
        mesh=mesh,
        scratch_types=[
            pltpu.VMEM((2, K), jnp.int32),
            pltpu.VMEM((2, K), jnp.int32),
            pltpu.VMEM((K,), jnp.int32),
            pltpu.VMEM((K,), jnp.int32),
            pltpu.VMEM((K,), jnp.int32),
            pltpu.VMEM((K,), jnp.int32),
            pltpu.VMEM((K,), jnp.float32),
            pltpu.VMEM((K,), jnp.float32),
            pltpu.VMEM((K,), jnp.float32),
            pltpu.VMEM((NROW_T,), jnp.float32),
            pltpu.VMEM((128,), jnp.int32),
            pltpu.VMEM((128,), jnp.int32),
            pltpu.VMEM((128,), jnp.float32),
            pltpu.VMEM_SHARED((N2,), jnp.float32),
            pltpu.VMEM_SHARED((N2,), jnp.float32),
            pltpu.VMEM_SHARED((N2,), jnp.float32),
            pltpu.SemaphoreType.DMA,
            pltpu.SemaphoreType.DMA,
            pltpu.SemaphoreType.DMA,
            pltpu.SemaphoreType.DMA,
            pltpu.SemaphoreType.DMA,
        ],
    )
    return f(ei, p, z, ones)


def _combine_body(s0_ref, s1_ref, c0_ref, c1_ref, r_ref, scal_ref, out_ref):
    ssum = s0_ref[...] + s1_ref[...]
    cnt = c0_ref[...] + c1_ref[...]
    mean = ssum / jnp.maximum(cnt, 1.0)
    h = mean + scal_ref[0, 0] + r_ref[...]
    h = jnp.where(h > 0, h, jnp.exp(h) - 1.0)
    out_ref[...] = h * scal_ref[0, 1] + scal_ref[0, 2]


def _combine(s2m, c2m, r2d, scal):
    spec0 = pl.BlockSpec((RB, 128), lambda i: (i, 0))
    spec1 = pl.BlockSpec((RB, 128), lambda i: (i + R // RB, 0))
    return pl.pallas_call(
        _combine_body,
        grid=(R // RB,),
        in_specs=[
            spec0, spec1, spec0, spec1, spec0,
            pl.BlockSpec(memory_space=pltpu.SMEM),
        ],
        out_specs=pl.BlockSpec((RB, 128), lambda i: (i, 0)),
        out_shape=jax.ShapeDtypeStruct((R, 128), jnp.float32),
    )(s2m, s2m, c2m, c2m, r2d, scal)


def kernel(x, edge_index, edge_weight, W_l, b_l, W_r, W_out, b_out):
    ei = edge_index.astype(jnp.int32)
    w2 = jnp.concatenate([W_l, W_r], axis=0).T
    p2d, r2d = _proj(x.T, w2)
    z = jnp.zeros((NROW_T,), jnp.float32)
    ones = jnp.ones((K,), jnp.float32)
    s2, c2 = _segment_acc(ei, p2d.reshape(N2), z, ones)
    scal = jnp.concatenate([b_l, W_out[0], b_out]).reshape(1, 3)
    out2d = _combine(s2.reshape(2 * R, 128), c2.reshape(2 * R, 128),
                     r2d, scal)
    return out2d.reshape(N2, 1)[:N_NODES]


# speedup vs baseline: 174.8238x; 1.1532x over previous
import jax
import jax.numpy as jnp
from jax import lax
from jax.experimental import pallas as pl
from jax.experimental.pallas import tpu as pltpu
from jax.experimental.pallas import tpu_sc as plsc

N_NODES = 100000
N_EDGES = 3200000
D_IN = 17

BLK = 14336
GRID = 7
N2 = BLK * GRID
R = N2 // 128
RB = 112
NC, NS = 2, 16
NW = NC * NS
EP_TILE = 99968
K = 9088
NCHUNK = EP_TILE // K
TAIL_BASE = NW * EP_TILE
TAIL_N = N_EDGES - TAIL_BASE
NROW_T = N2 // NS


def _proj_body(xt_ref, w2_ref, p_ref, r_ref):
    pr = lax.dot_general(w2_ref[...], xt_ref[...],
                         dimension_numbers=(((0,), (0,)), ((), ())),
                         preferred_element_type=jnp.float32)
    p_ref[...] = pr[0].reshape(BLK // 128, 128)
    r_ref[...] = pr[1].reshape(BLK // 128, 128)


def _proj(xt, w2):
    return pl.pallas_call(
        _proj_body,
        grid=(GRID,),
        in_specs=[
            pl.BlockSpec((D_IN, BLK), lambda i: (0, i)),
            pl.BlockSpec((D_IN, 2), lambda i: (0, 0)),
        ],
        out_specs=[
            pl.BlockSpec((BLK // 128, 128), lambda i: (i, 0)),
            pl.BlockSpec((BLK // 128, 128), lambda i: (i, 0)),
        ],
        out_shape=[
            jax.ShapeDtypeStruct((R, 128), jnp.float32),
            jax.ShapeDtypeStruct((R, 128), jnp.float32),
        ],
    )(xt, w2)


def _deinterleave(eib, src_v, dst_v, n):
    def body(i, _):
        off = pl.ds(pl.multiple_of(i * 16, 16), 16)
        src_v[off] = eib[0, off]
        dst_v[off] = eib[1, off]
        return 0

    lax.fori_loop(0, n // 16, body, 0)


def _sc_body(ei_hbm, p_hbm, z_hbm, ones_hbm, s_out, c_out,
             ei_v0, ei_v1, src_v0, src_v1, dst_v0, dst_v1, vals_v0,
             vals_v1, ones_v, stage_v, src_t, dst_t, vals_t,
             p_s, s_s, c_s, sem_i0, sem_i1, sem_g, sem_s, sem_c):
    c = lax.axis_index("c")
    s = lax.axis_index("s")
    wid = c * NS + s
    row = s * NROW_T
    ei_bufs = (ei_v0, ei_v1)
    src_bufs = (src_v0, src_v1)
    dst_bufs = (dst_v0, dst_v1)
    vals_bufs = (vals_v0, vals_v1)
    sems = (sem_i0, sem_i1)

    def idx_start(j):
        base = wid * EP_TILE + j * K
        return pltpu.async_copy(ei_hbm.at[:, pl.ds(base, K)],
                                ei_bufs[j % 2], sems[j % 2])

    d_idx = idx_start(0)

    pltpu.sync_copy(z_hbm, stage_v)
    pltpu.sync_copy(stage_v, s_s.at[pl.ds(row, NROW_T)])
    pltpu.sync_copy(stage_v, c_s.at[pl.ds(row, NROW_T)])
    pltpu.sync_copy(p_hbm.at[pl.ds(row, NROW_T)], stage_v)
    pltpu.sync_copy(stage_v, p_s.at[pl.ds(row, NROW_T)])
    pltpu.sync_copy(ones_hbm, ones_v)
    plsc.subcore_barrier()

    d_ss = d_sc = None
    for j in range(NCHUNK):
        b = j % 2
        d_idx.wait()
        if j + 1 < NCHUNK:
            d_idx = idx_start(j + 1)
        if j >= 2:
            d_ss2.wait()
            d_sc2.wait()
        _deinterleave(ei_bufs[b], src_bufs[b], dst_bufs[b], K)
        pltpu.async_copy(p_s.at[src_bufs[b]], vals_bufs[b], sem_g).wait()
        d_ss2, d_sc2 = d_ss, d_sc
        d_ss = pltpu.async_copy(vals_bufs[b], s_s.at[dst_bufs[b]],
                                sem_s, add=True)
        d_sc = pltpu.async_copy(ones_v, c_s.at[dst_bufs[b]],
                                sem_c, add=True)
    d_ss2.wait()
    d_sc2.wait()
    d_ss.wait()
    d_sc.wait()

    @pl.when(wid == NW - 1)
    def _tail():
        pltpu.sync_copy(ei_hbm.at[:, pl.ds(TAIL_BASE, TAIL_N)],
                        ei_v0.at[:, pl.ds(0, TAIL_N)])
        _deinterleave(ei_v0, src_t, dst_t, TAIL_N)
        pltpu.async_copy(p_s.at[src_t], vals_t, sem_g).wait()
        pltpu.sync_copy(vals_t, s_s.at[dst_t], add=True)
        pltpu.sync_copy(ones_v.at[pl.ds(0, TAIL_N)], c_s.at[dst_t],
                        add=True)

    plsc.subcore_barrier()
    off = c * N2 + row
    pltpu.sync_copy(s_s.at[pl.ds(row, NROW_T)], stage_v)
    pltpu.sync_copy(stage_v, s_out.at[pl.ds(off, NROW_T)])
    pltpu.sync_copy(c_s.at[pl.ds(row, NROW_T)], stage_v)
    pltpu.sync_copy(stage_v, c_out.at[pl.ds(off, NROW_T)])


def _segment_acc(ei, p, z, ones):
    mesh = plsc.VectorSubcoreMesh(core_axis_name="c", subcore_axis_name="s",
                                  num_cores=NC, num_subcores=NS)
    f = pl.kernel(
        _sc_body,
        out_type=[
            jax.ShapeDtypeStruct((NC * N2,), jnp.float32),
            jax.ShapeDtypeStruct((NC * N2,), jnp.float32),
        ],
        mesh=mesh,
        scratch_types=[
            pltpu.VMEM((2, K), jnp.int32),
            pltpu.VMEM((2, K), jnp.int32),
            pltpu.VMEM((K,), jnp.int32),
            pltpu.VMEM((K,), jnp.int32),
            pltpu.VMEM((K,), jnp.int32),
            pltpu.VMEM((K,), jnp.int32),
            pltpu.VMEM((K,), jnp.float32),
            pltpu.VMEM((K,), jnp.float32),
            pltpu.VMEM((K,), jnp.float32),
            pltpu.VMEM((NROW_T,), jnp.float32),
            pltpu.VMEM((TAIL_N,), jnp.int32),
            pltpu.VMEM((TAIL_N,), jnp.int32),
            pltpu.VMEM((TAIL_N,), jnp.float32),
            pltpu.VMEM_SHARED((N2,), jnp.float32),
            pltpu.VMEM_SHARED((N2,), jnp.float32),
            pltpu.VMEM_SHARED((N2,), jnp.float32),
            pltpu.SemaphoreType.DMA,
            pltpu.SemaphoreType.DMA,
            pltpu.SemaphoreType.DMA,
            pltpu.SemaphoreType.DMA,
            pltpu.SemaphoreType.DMA,
        ],
    )
    return f(ei, p, z, ones)


def _combine_body(s0_ref, s1_ref, c0_ref, c1_ref, r_ref, scal_ref, out_ref):
    ssum = s0_ref[...] + s1_ref[...]
    cnt = c0_ref[...] + c1_ref[...]
    mean = ssum / jnp.maximum(cnt, 1.0)
    h = mean + scal_ref[0, 0] + r_ref[...]
    h = jnp.where(h > 0, h, jnp.exp(h) - 1.0)
    out_ref[...] = h * scal_ref[0, 1] + scal_ref[0, 2]


def _combine(s2m, c2m, r2d, scal):
    spec0 = pl.BlockSpec((RB, 128), lambda i: (i, 0))
    spec1 = pl.BlockSpec((RB, 128), lambda i: (i + R // RB, 0))
    return pl.pallas_call(
        _combine_body,
        grid=(R // RB,),
        in_specs=[
            spec0, spec1, spec0, spec1, spec0,
            pl.BlockSpec(memory_space=pltpu.SMEM),
        ],
        out_specs=pl.BlockSpec((RB, 128), lambda i: (i, 0)),
        out_shape=jax.ShapeDtypeStruct((R, 128), jnp.float32),
    )(s2m, s2m, c2m, c2m, r2d, scal)


def kernel(x, edge_index, edge_weight, W_l, b_l, W_r, W_out, b_out):
    ei = edge_index.astype(jnp.int32)
    w2 = jnp.concatenate([W_l, W_r], axis=0).T
    p2d, r2d = _proj(x.T, w2)
    z = jnp.zeros((NROW_T,), jnp.float32)
    ones = jnp.ones((K,), jnp.float32)
    s2, c2 = _segment_acc(ei, p2d.reshape(N2), z, ones)
    scal = jnp.concatenate([b_l, W_out[0], b_out]).reshape(1, 3)
    out2d = _combine(s2.reshape(2 * R, 128), c2.reshape(2 * R, 128),
                     r2d, scal)
    return out2d.reshape(N2, 1)[:N_NODES]


# speedup vs baseline: 177.7094x; 1.0165x over previous
import jax
import jax.numpy as jnp
from jax import lax
from jax.experimental import pallas as pl
from jax.experimental.pallas import tpu as pltpu
from jax.experimental.pallas import tpu_sc as plsc

N_NODES = 100000
N_EDGES = 3200000
D_IN = 17

BLK = 14336
GRID = 7
N2 = BLK * GRID
R = N2 // 128
RB = 112
NC, NS = 2, 16
NW = NC * NS
EP_TILE = 99968
K = 9088
NCHUNK = EP_TILE // K
TAIL_BASE = NW * EP_TILE
TAIL_N = N_EDGES - TAIL_BASE
NROW_T = N2 // NS


def _proj_body(xt_ref, w2_ref, p_ref, r_ref):
    pr = lax.dot_general(w2_ref[...], xt_ref[...],
                         dimension_numbers=(((0,), (0,)), ((), ())),
                         preferred_element_type=jnp.float32)
    p_ref[...] = pr[0].reshape(BLK // 128, 128)
    r_ref[...] = pr[1].reshape(BLK // 128, 128)


def _proj(xt, w2):
    return pl.pallas_call(
        _proj_body,
        grid=(GRID,),
        in_specs=[
            pl.BlockSpec((D_IN, BLK), lambda i: (0, i)),
            pl.BlockSpec((D_IN, 2), lambda i: (0, 0)),
        ],
        out_specs=[
            pl.BlockSpec((BLK // 128, 128), lambda i: (i, 0)),
            pl.BlockSpec((BLK // 128, 128), lambda i: (i, 0)),
        ],
        out_shape=[
            jax.ShapeDtypeStruct((R, 128), jnp.float32),
            jax.ShapeDtypeStruct((R, 128), jnp.float32),
        ],
    )(xt, w2)


def _deinterleave(eib, src_v, dst_v, n):
    def body(i, _):
        for u in range(8):
            off = pl.ds(pl.multiple_of(i * 128 + u * 16, 16), 16)
            src_v[off] = eib[0, off]
            dst_v[off] = eib[1, off]
        return 0

    lax.fori_loop(0, n // 128, body, 0)


def _sc_body(ei_hbm, p_hbm, z_hbm, ones_hbm, s_out, c_out,
             ei_v0, ei_v1, src_v0, src_v1, dst_v0, dst_v1, vals_v0,
             vals_v1, ones_v, stage_v, src_t, dst_t, vals_t,
             p_s, s_s, c_s, sem_i0, sem_i1, sem_g, sem_s, sem_c):
    c = lax.axis_index("c")
    s = lax.axis_index("s")
    wid = c * NS + s
    row = s * NROW_T
    ei_bufs = (ei_v0, ei_v1)
    src_bufs = (src_v0, src_v1)
    dst_bufs = (dst_v0, dst_v1)
    vals_bufs = (vals_v0, vals_v1)
    sems = (sem_i0, sem_i1)

    def idx_start(j):
        base = wid * EP_TILE + j * K
        return pltpu.async_copy(ei_hbm.at[:, pl.ds(base, K)],
                                ei_bufs[j % 2], sems[j % 2])

    d_idx = idx_start(0)

    pltpu.sync_copy(z_hbm, stage_v)
    pltpu.sync_copy(stage_v, s_s.at[pl.ds(row, NROW_T)])
    pltpu.sync_copy(stage_v, c_s.at[pl.ds(row, NROW_T)])
    pltpu.sync_copy(p_hbm.at[pl.ds(row, NROW_T)], stage_v)
    pltpu.sync_copy(stage_v, p_s.at[pl.ds(row, NROW_T)])
    pltpu.sync_copy(ones_hbm, ones_v)
    plsc.subcore_barrier()

    d_ss = d_sc = None
    for j in range(NCHUNK):
        b = j % 2
        d_idx.wait()
        if j + 1 < NCHUNK:
            d_idx = idx_start(j + 1)
        if j >= 2:
            d_ss2.wait()
            d_sc2.wait()
        _deinterleave(ei_bufs[b], src_bufs[b], dst_bufs[b], K)
        pltpu.async_copy(p_s.at[src_bufs[b]], vals_bufs[b], sem_g).wait()
        d_ss2, d_sc2 = d_ss, d_sc
        d_ss = pltpu.async_copy(vals_bufs[b], s_s.at[dst_bufs[b]],
                                sem_s, add=True)
        d_sc = pltpu.async_copy(ones_v, c_s.at[dst_bufs[b]],
                                sem_c, add=True)
    d_ss2.wait()
    d_sc2.wait()
    d_ss.wait()
    d_sc.wait()

    @pl.when(wid == NW - 1)
    def _tail():
        pltpu.sync_copy(ei_hbm.at[:, pl.ds(TAIL_BASE, TAIL_N)],
                        ei_v0.at[:, pl.ds(0, TAIL_N)])
        _deinterleave(ei_v0, src_t, dst_t, TAIL_N)
        pltpu.async_copy(p_s.at[src_t], vals_t, sem_g).wait()
        pltpu.sync_copy(vals_t, s_s.at[dst_t], add=True)
        pltpu.sync_copy(ones_v.at[pl.ds(0, TAIL_N)], c_s.at[dst_t],
                        add=True)

    plsc.subcore_barrier()
    off = c * N2 + row
    pltpu.sync_copy(s_s.at[pl.ds(row, NROW_T)], stage_v)
    pltpu.sync_copy(stage_v, s_out.at[pl.ds(off, NROW_T)])
    pltpu.sync_copy(c_s.at[pl.ds(row, NROW_T)], stage_v)
    pltpu.sync_copy(stage_v, c_out.at[pl.ds(off, NROW_T)])


def _segment_acc(ei, p, z, ones):
    mesh = plsc.VectorSubcoreMesh(core_axis_name="c", subcore_axis_name="s",
                                  num_cores=NC, num_subcores=NS)
    f = pl.kernel(
        _sc_body,
        out_type=[
            jax.ShapeDtypeStruct((NC * N2,), jnp.float32),
            jax.ShapeDtypeStruct((NC * N2,), jnp.float32),
        ],
        mesh=mesh,
        scratch_types=[
            pltpu.VMEM((2, K), jnp.int32),
            pltpu.VMEM((2, K), jnp.int32),
            pltpu.VMEM((K,), jnp.int32),
            pltpu.VMEM((K,), jnp.int32),
            pltpu.VMEM((K,), jnp.int32),
            pltpu.VMEM((K,), jnp.int32),
            pltpu.VMEM((K,), jnp.float32),
            pltpu.VMEM((K,), jnp.float32),
            pltpu.VMEM((K,), jnp.float32),
            pltpu.VMEM((NROW_T,), jnp.float32),
            pltpu.VMEM((TAIL_N,), jnp.int32),
            pltpu.VMEM((TAIL_N,), jnp.int32),
            pltpu.VMEM((TAIL_N,), jnp.float32),
            pltpu.VMEM_SHARED((N2,), jnp.float32),
            pltpu.VMEM_SHARED((N2,), jnp.float32),
            pltpu.VMEM_SHARED((N2,), jnp.float32),
            pltpu.SemaphoreType.DMA,
            pltpu.SemaphoreType.DMA,
            pltpu.SemaphoreType.DMA,
            pltpu.SemaphoreType.DMA,
            pltpu.SemaphoreType.DMA,
        ],
    )
    return f(ei, p, z, ones)


def _combine_body(s0_ref, s1_ref, c0_ref, c1_ref, r_ref, scal_ref, out_ref):
    ssum = s0_ref[...] + s1_ref[...]
    cnt = c0_ref[...] + c1_ref[...]
    mean = ssum / jnp.maximum(cnt, 1.0)
    h = mean + scal_ref[0, 0] + r_ref[...]
    h = jnp.where(h > 0, h, jnp.exp(h) - 1.0)
    out_ref[...] = h * scal_ref[0, 1] + scal_ref[0, 2]


def _combine(s2m, c2m, r2d, scal):
    spec0 = pl.BlockSpec((RB, 128), lambda i: (i, 0))
    spec1 = pl.BlockSpec((RB, 128), lambda i: (i + R // RB, 0))
    return pl.pallas_call(
        _combine_body,
        grid=(R // RB,),
        in_specs=[
            spec0, spec1, spec0, spec1, spec0,
            pl.BlockSpec(memory_space=pltpu.SMEM),
        ],
        out_specs=pl.BlockSpec((RB, 128), lambda i: (i, 0)),
        out_shape=jax.ShapeDtypeStruct((R, 128), jnp.float32),
    )(s2m, s2m, c2m, c2m, r2d, scal)


def kernel(x, edge_index, edge_weight, W_l, b_l, W_r, W_out, b_out):
    ei = edge_index.astype(jnp.int32)
    w2 = jnp.concatenate([W_l, W_r], axis=0).T
    p2d, r2d = _proj(x.T, w2)
    z = jnp.zeros((NROW_T,), jnp.float32)
    ones = jnp.ones((K,), jnp.float32)
    s2, c2 = _segment_acc(ei, p2d.reshape(N2), z, ones)
    scal = jnp.concatenate([b_l, W_out[0], b_out]).reshape(1, 3)
    out2d = _combine(s2.reshape(2 * R, 128), c2.reshape(2 * R, 128),
                     r2d, scal)
    return out2d.reshape(N2, 1)[:N_NODES]
